# Initial kernel scaffold; baseline (speedup 1.0000x reference)
import jax
import jax.numpy as jnp
from jax.experimental import pallas as pl


def kernel(tokenized_prompts, token_embedding):
    raise NotImplementedError("write your pallas kernel here")


# speedup vs baseline: 1.1423x; 1.1423x over previous
import functools

import jax
import jax.numpy as jnp
from jax import lax
from jax.experimental import pallas as pl
from jax.experimental.pallas import tpu as pltpu
from jax.experimental.pallas import tpu_sc as plsc

_LANES = 16


@functools.lru_cache(maxsize=None)
def _make_sc_kernel(n_cls, seq, seqp, dim, num_cores, n_workers, cpw):
    nchunk = dim // _LANES
    nidx = seqp // _LANES
    inv_seq = 1.0 / seq
    mesh = plsc.VectorSubcoreMesh(core_axis_name="c", subcore_axis_name="s")

    @functools.partial(
        pl.kernel,
        mesh=mesh,
        out_type=(
            jax.ShapeDtypeStruct((n_cls, seq, dim), jnp.float32),
            jax.ShapeDtypeStruct((n_cls, dim), jnp.float32),
        ),
        scratch_types=[
            pltpu.VMEM((8, seqp), jnp.int32),
            pltpu.VMEM((seqp,), jnp.int32),
            pltpu.VMEM((seqp,), jnp.int32),
            pltpu.VMEM((seqp, dim), jnp.float32),
            pltpu.VMEM((seqp, dim), jnp.float32),
            pltpu.VMEM((seq, dim), jnp.float32),
            pltpu.VMEM((8, dim), jnp.float32),
            pltpu.SemaphoreType.DMA,
            pltpu.SemaphoreType.DMA,
        ],
    )
    def kfn(idx_hbm, table_hbm, emb_hbm, mean_hbm,
            idx_v, row0, row1, buf0, buf1, stage, mean8, sem0, sem1):
        wid = lax.axis_index("s") * num_cores + lax.axis_index("c")
        base = wid * cpw
        n = jnp.minimum(cpw, n_cls - base)

        pltpu.sync_copy(idx_hbm.at[pl.ds(base, 8)], idx_v)

        def fill_and_start(cl, row, buf, sem):
            for j in range(nidx):
                sl = pl.ds(_LANES * j, _LANES)
                row[sl] = idx_v[cl % 8, sl]
            pltpu.async_copy(table_hbm.at[row], buf, sem)

        def wait(row, buf, sem):
            pltpu.make_async_copy(table_hbm.at[row], buf, sem).wait()

        def process(cl, buf):
            accs = []
            for j in range(nchunk):
                sl = pl.ds(_LANES * j, _LANES)
                v = buf[0, sl]
                stage[0, sl] = v
                accs.append(v)

            def rbody(r, accs):
                out = []
                for j in range(nchunk):
                    sl = pl.ds(_LANES * j, _LANES)
                    v = buf[r, sl]
                    stage[r, sl] = v
                    out.append(accs[j] + v)
                return tuple(out)

            accs = lax.fori_loop(1, seq, rbody, tuple(accs))
            m = cl % 8
            for j in range(nchunk):
                sl = pl.ds(_LANES * j, _LANES)
                mean8[m, sl] = accs[j] * inv_seq
            pltpu.sync_copy(stage, emb_hbm.at[base + cl])

        fill_and_start(0, row0, buf0, sem0)
        fill_and_start(1, row1, buf1, sem1)

        def pair_body(p, carry):
            c0 = 2 * p

            @pl.when(jnp.logical_and((p & 3) == 3, ((p >> 2) + 1) * 8 < n))
            def _():
                pltpu.sync_copy(
                    idx_hbm.at[pl.ds(base + ((p >> 2) + 1) * 8, 8)], idx_v)

            wait(row0, buf0, sem0)
            process(c0, buf0)

            @pl.when(c0 + 2 < n)
            def _():
                fill_and_start(c0 + 2, row0, buf0, sem0)

            wait(row1, buf1, sem1)
            process(c0 + 1, buf1)

            @pl.when(c0 + 3 < n)
            def _():
                fill_and_start(c0 + 3, row1, buf1, sem1)

            @pl.when((p & 3) == 3)
            def _():
                pltpu.sync_copy(mean8,
                                mean_hbm.at[pl.ds(base + (p >> 2) * 8, 8)])

            return carry

        lax.fori_loop(0, n // 2, pair_body, 0)

    return kfn


def kernel(tokenized_prompts, token_embedding):
    n_cls, seq = tokenized_prompts.shape
    _, dim = token_embedding.shape
    info = plsc.get_sparse_core_info()
    n_workers = info.num_cores * info.num_subcores
    cpw = -(-n_cls // n_workers)
    seqp = -(-seq // _LANES) * _LANES
    idx = tokenized_prompts.astype(jnp.int32)
    idx_pad = jnp.pad(idx, ((0, 0), (0, seqp - seq)))
    table = token_embedding.astype(jnp.float32)
    emb, mean = _make_sc_kernel(
        n_cls, seq, seqp, dim, info.num_cores, n_workers, cpw)(idx_pad, table)
    return emb, mean


# speedup vs baseline: 1.1834x; 1.0360x over previous
import functools

import jax
import jax.numpy as jnp
from jax import lax
from jax.experimental import pallas as pl
from jax.experimental.pallas import tpu as pltpu
from jax.experimental.pallas import tpu_sc as plsc

_LANES = 16
_MAIN = 72


@functools.lru_cache(maxsize=None)
def _make_sc_kernel(n_cls, seq, seqp, dim, num_cores, n_workers, cpw):
    nchunk = dim // _LANES
    ntail = seq - _MAIN
    inv_seq = 1.0 / seq
    mesh = plsc.VectorSubcoreMesh(core_axis_name="c", subcore_axis_name="s")

    @functools.partial(
        pl.kernel,
        mesh=mesh,
        out_type=(
            jax.ShapeDtypeStruct((n_cls, seq, dim), jnp.float32),
            jax.ShapeDtypeStruct((n_cls, dim), jnp.float32),
        ),
        scratch_types=[
            pltpu.VMEM((_MAIN,), jnp.int32),
            pltpu.VMEM((_MAIN,), jnp.int32),
            pltpu.VMEM((8,), jnp.int32),
            pltpu.VMEM((8,), jnp.int32),
            pltpu.VMEM((seq, dim), jnp.float32),
            pltpu.VMEM((seq, dim), jnp.float32),
            pltpu.VMEM((8, dim), jnp.float32),
            pltpu.VMEM((8, dim), jnp.float32),
            pltpu.VMEM((8, dim), jnp.float32),
            pltpu.SemaphoreType.DMA,
            pltpu.SemaphoreType.DMA,
            pltpu.SemaphoreType.DMA,
            pltpu.SemaphoreType.DMA,
        ],
    )
    def kfn(idxf_hbm, table_hbm, emb_hbm, mean_hbm,
            idx72_0, idx72_1, idx8_0, idx8_1, buf0, buf1, tail0, tail1,
            mean8, semg0, semg1, semw0, semw1):
        wid = lax.axis_index("s") * num_cores + lax.axis_index("c")
        base = wid * cpw
        n = jnp.minimum(cpw, n_cls - base)

        idx72 = (idx72_0, idx72_1)
        idx8 = (idx8_0, idx8_1)
        buf = (buf0, buf1)
        tail = (tail0, tail1)
        semg = (semg0, semg1)
        semw = (semw0, semw1)

        def stage_idx_and_tail(c, b):
            o = (base + c) * seqp
            pltpu.sync_copy(idxf_hbm.at[pl.ds(o, _MAIN)], idx72[b])
            pltpu.sync_copy(idxf_hbm.at[pl.ds(o + _MAIN, 8)], idx8[b])
            pltpu.async_copy(table_hbm.at[idx8[b]], tail[b], semg[b])

        def start_main(b):
            pltpu.async_copy(table_hbm.at[idx72[b]],
                             buf[b].at[pl.ds(0, _MAIN)], semg[b])

        def wait_gathers(b):
            pltpu.make_async_copy(table_hbm.at[idx72[b]],
                                  buf[b].at[pl.ds(0, _MAIN)], semg[b]).wait()
            pltpu.make_async_copy(table_hbm.at[idx8[b]], tail[b],
                                  semg[b]).wait()

        def wait_write(b):
            pltpu.make_async_copy(buf[b], emb_hbm.at[base], semw[b]).wait()

        def process(c, b):
            wait_gathers(b)
            for r in range(ntail):
                for j in range(nchunk):
                    sl = pl.ds(_LANES * j, _LANES)
                    buf[b][_MAIN + r, sl] = tail[b][r, sl]

            accs = []
            for j in range(nchunk):
                sl = pl.ds(_LANES * j, _LANES)
                accs.append(buf[b][0, sl])

            def rbody(r, accs):
                return tuple(accs[j] + buf[b][r, pl.ds(_LANES * j, _LANES)]
                             for j in range(nchunk))

            accs = lax.fori_loop(1, seq, rbody, tuple(accs))
            m = c % 8
            for j in range(nchunk):
                mean8[m, pl.ds(_LANES * j, _LANES)] = accs[j] * inv_seq
            pltpu.async_copy(buf[b], emb_hbm.at[base + c], semw[b])

        stage_idx_and_tail(0, 0)
        start_main(0)
        stage_idx_and_tail(1, 1)
        start_main(1)

        def pair_body(p, carry):
            c0 = 2 * p
            c1 = c0 + 1

            process(c0, 0)

            @pl.when(c0 + 2 < n)
            def _():
                stage_idx_and_tail(c0 + 2, 0)

            @pl.when(p > 0)
            def _():
                wait_write(1)
                start_main(1)

            process(c1, 1)

            @pl.when((p & 3) == 3)
            def _():
                pltpu.sync_copy(mean8,
                                mean_hbm.at[pl.ds(base + (p >> 2) * 8, 8)])

            @pl.when(c1 + 2 < n)
            def _():
                stage_idx_and_tail(c1 + 2, 1)

            @pl.when(c0 + 2 < n)
            def _():
                wait_write(0)
                start_main(0)

            return carry

        lax.fori_loop(0, n // 2, pair_body, 0)
        wait_write(0)
        wait_write(1)

    return kfn


def kernel(tokenized_prompts, token_embedding):
    n_cls, seq = tokenized_prompts.shape
    _, dim = token_embedding.shape
    info = plsc.get_sparse_core_info()
    n_workers = info.num_cores * info.num_subcores
    cpw = -(-n_cls // n_workers)
    seqp = -(-seq // _LANES) * _LANES
    idx = tokenized_prompts.astype(jnp.int32)
    idx_flat = jnp.pad(idx, ((0, 0), (0, seqp - seq))).reshape(-1)
    table = token_embedding.astype(jnp.float32)
    emb, mean = _make_sc_kernel(
        n_cls, seq, seqp, dim, info.num_cores, n_workers, cpw)(idx_flat, table)
    return emb, mean
